# baseline (speedup 1.0000x reference)
import functools

import jax
import jax.numpy as jnp
from jax import lax
from jax.experimental import pallas as pl
from jax.experimental.pallas import tpu as pltpu
from jax.experimental.pallas import tpu_sc as plsc

_B = 16384
_L = 20
_NC = 2
_NS = 16
_NW = _NC * _NS
_LANES = 16
_SAMPLES_PER_W = _B // _NW
_TOL = 0.5

_mesh = plsc.VectorSubcoreMesh(
    core_axis_name="c", subcore_axis_name="s",
    num_cores=_NC, num_subcores=_NS)


@functools.partial(
    pl.kernel,
    out_type=jax.ShapeDtypeStruct((_NW, 2, _LANES), jnp.float32),
    mesh=_mesh,
    compiler_params=pltpu.CompilerParams(needs_layout_passes=False),
    scratch_types=[
        pltpu.VMEM((_L, _SAMPLES_PER_W), jnp.int32),
        pltpu.VMEM((_L, _SAMPLES_PER_W), jnp.float32),
        pltpu.VMEM((120,), jnp.float32),
        pltpu.VMEM((256,), jnp.float32),
        pltpu.VMEM((2, _LANES), jnp.float32),
        pltpu.SemaphoreType.DMA,
        pltpu.SemaphoreType.DMA,
    ],
)
def _sc_charge_loss(pw_hbm, frac_hbm, table_hbm, out_hbm,
                    pw_v, frac_v, table_v, t256_v, out_v, sem1, sem2):
    wid = lax.axis_index("s") * _NC + lax.axis_index("c")
    base = wid * _SAMPLES_PER_W
    cp1 = pltpu.async_copy(pw_hbm.at[:, pl.ds(base, _SAMPLES_PER_W)],
                           pw_v, sem1)
    cp2 = pltpu.async_copy(frac_hbm.at[:, pl.ds(base, _SAMPLES_PER_W)],
                           frac_v, sem2)
    pltpu.sync_copy(table_hbm, table_v)

    iota = lax.iota(jnp.int32, _LANES)
    for k in range(256 // _LANES):
        i = k * _LANES + iota
        idx = jnp.minimum(i & 127, 119)
        mf = (i >> 7).astype(jnp.float32)
        t256_v[pl.ds(k * _LANES, _LANES)] = (
            plsc.load_gather(table_v, [idx]) * mf)

    cp1.wait()
    cp2.wait()

    def body(g, carry):
        loss_acc, abs_acc = carry
        accs = []
        for half in range(2):
            c0 = (g * 2 + half) * _LANES
            tcs = [jnp.zeros((_LANES,), jnp.float32) for _ in range(4)]
            for l in range(0, _L, 4):
                for q in range(4):
                    w = pw_v[l + q, pl.ds(c0, _LANES)]
                    f = frac_v[l + q, pl.ds(c0, _LANES)]
                    tcs[q] = tcs[q] + f * plsc.load_gather(t256_v, [w])
            tc = (tcs[0] + tcs[1]) + (tcs[2] + tcs[3])
            a = jnp.abs(tc)
            ex = jnp.maximum(a - _TOL, 0.0)
            e2 = jnp.exp(2.0 * ex)
            t = 1.0 - 2.0 / (e2 + 1.0)
            accs.append((t, a))
        loss_acc = loss_acc + accs[0][0] + accs[1][0]
        abs_acc = abs_acc + accs[0][1] + accs[1][1]
        return loss_acc, abs_acc

    zero = jnp.zeros((_LANES,), jnp.float32)
    loss_acc, abs_acc = lax.fori_loop(
        0, _SAMPLES_PER_W // (2 * _LANES), body, (zero, zero))

    out_v[0, :] = loss_acc * (1.0 / _B)
    out_v[1, :] = abs_acc * (1.0 / _B)
    pltpu.sync_copy(out_v, out_hbm.at[wid])


def kernel(element_indices, element_fractions, element_mask, oxidation_states):
    pw = (element_indices.astype(jnp.int32)
          | (element_mask.astype(jnp.int32) << 7)).T
    ef = element_fractions.T
    partials = _sc_charge_loss(pw, ef, oxidation_states)
    charge_balance_loss = jnp.sum(partials[:, 0, :])
    mean_charge_imbalance = jnp.sum(partials[:, 1, :])
    return (charge_balance_loss, mean_charge_imbalance)
